# Initial kernel scaffold; baseline (speedup 1.0000x reference)
import jax
import jax.numpy as jnp
from jax.experimental import pallas as pl


def kernel(idx, raw):
    raise NotImplementedError("write your pallas kernel here")


# speedup vs baseline: 3.7598x; 3.7598x over previous
import functools

import jax
import jax.numpy as jnp
from jax import lax
from jax.experimental import pallas as pl
from jax.experimental.pallas import tpu as pltpu
from jax.experimental.pallas import tpu_sc as plsc

_VOCAB = 100000
_EMBED = 64
_B = 4096
_S = 50
_NC = 2
_NS = 16
_NW = _NC * _NS
_ROWS_PER_W = _B // _NW
_CHUNK = 16
_NCHUNK = _ROWS_PER_W // _CHUNK


def _softplus_body(raw_ref, w_ref):
    w_ref[...] = jax.nn.softplus(raw_ref[...])


def _softplus_table(raw):
    blk = 4000
    return pl.pallas_call(
        _softplus_body,
        out_shape=jax.ShapeDtypeStruct((_VOCAB, _EMBED), jnp.float32),
        grid=(_VOCAB // blk,),
        in_specs=[pl.BlockSpec((blk, _EMBED), lambda i: (i, 0))],
        out_specs=pl.BlockSpec((blk, _EMBED), lambda i: (i, 0)),
    )(raw)


def _make_gather():
    mesh = plsc.VectorSubcoreMesh(
        core_axis_name="c", subcore_axis_name="s",
        num_cores=_NC, num_subcores=_NS)

    @functools.partial(
        pl.kernel,
        out_type=jax.ShapeDtypeStruct((_B, _S, _EMBED), jnp.float32),
        mesh=mesh,
        compiler_params=pltpu.CompilerParams(use_tc_tiling_on_sc=False),
        scratch_types=[
            pltpu.VMEM((_ROWS_PER_W, _S), jnp.int32),
            pltpu.VMEM((_CHUNK, _S, _EMBED), jnp.float32),
            pltpu.SemaphoreType.DMA,
        ],
    )
    def gather_kernel(table_hbm, idx_hbm, out_hbm, idx_v, rows_v, sem):
        wid = lax.axis_index("s") * _NC + lax.axis_index("c")
        base = wid * _ROWS_PER_W
        pltpu.sync_copy(idx_hbm.at[pl.ds(base, _ROWS_PER_W)], idx_v)

        @pl.loop(0, _NCHUNK)
        def _chunk(c):
            copies = [
                pltpu.async_copy(
                    table_hbm.at[idx_v.at[c * _CHUNK + j]], rows_v.at[j], sem)
                for j in range(_CHUNK)
            ]
            for cp in copies:
                cp.wait()
            pltpu.sync_copy(rows_v, out_hbm.at[pl.ds(base + c * _CHUNK, _CHUNK)])

    return gather_kernel


def kernel(idx, raw):
    weight = _softplus_table(raw)
    return _make_gather()(weight, idx.astype(jnp.int32))
